# baseline (speedup 1.0000x reference)
import functools

import jax
import jax.numpy as jnp
from jax.experimental import pallas as pl
from jax.experimental.pallas import tpu as pltpu

_BF = 6400
_CH = 128


def _cs_kernel(xt_ref, bt_ref, gt_ref, wt_ref, bb_ref, out_ref,
               *, ns: int, oc: int, nn: int):
    qt = jnp.dot(wt_ref[...], xt_ref[0],
                 preferred_element_type=jnp.float32).astype(jnp.bfloat16)
    bf = qt.shape[1]
    nk = ns // nn
    for c in range(0, bf, _CH):
        w = min(_CH, bf - c)
        acc = None
        for n in range(nn):
            q0 = qt[(3 * n + 0) * oc:(3 * n + 1) * oc, c:c + w]
            d1 = qt[(3 * n + 1) * oc:(3 * n + 2) * oc, c:c + w]
            d2 = qt[(3 * n + 2) * oc:(3 * n + 3) * oc, c:c + w]
            accn = None
            for k in range(nk):
                s = n + nn * k
                x = (bt_ref[s:s + 1, c:c + w] * d1
                     + gt_ref[s:s + 1, c:c + w] * d2)
                accn = x if accn is None else jnp.maximum(accn, x)
            an = accn + q0
            acc = an if acc is None else jnp.maximum(acc, an)
        qc = qt[3 * nn * oc:(3 * nn + 1) * oc, c:c + w]
        r = jnp.maximum((acc - qc).astype(jnp.float32) + bb_ref[:, :w], 0.0)
        out_ref[0, c:c + w, :] = r.T


def kernel(ring_n, neighbor_corners, centers, alpha, beta, gamma, W, b):
    del ring_n, alpha
    m, f, nn = neighbor_corners.shape[:3]
    ns = beta.shape[1]
    oc = W.shape[0]
    nb = 3 * nn + 1
    x30 = jnp.concatenate(
        [neighbor_corners.reshape(m, f, nn * 9), centers], axis=-1)
    xt = x30.transpose(0, 2, 1)
    btt = beta.T.astype(jnp.bfloat16)
    gtt = gamma.T.astype(jnp.bfloat16)

    tm = jnp.zeros((nb, nb), jnp.float32)
    for n in range(nn):
        tm = tm.at[3 * n + 0, 3 * n + 0].set(1.0)
        tm = tm.at[3 * n + 0, 3 * n + 1].set(-1.0)
        tm = tm.at[3 * n + 0, 3 * n + 2].set(-1.0)
        tm = tm.at[3 * n + 1, 3 * n + 1].set(1.0)
        tm = tm.at[3 * n + 2, 3 * n + 2].set(1.0)
    tm = tm.at[nb - 1, nb - 1].set(1.0)
    wt = jnp.kron(tm, W.T).T
    bb = jnp.broadcast_to(b[:, None], (oc, _CH))

    grid = ((f + _BF - 1) // _BF, m)
    out_t = pl.pallas_call(
        functools.partial(_cs_kernel, ns=ns, oc=oc, nn=nn),
        grid=grid,
        in_specs=[
            pl.BlockSpec((1, 3 * nb, _BF), lambda fb, mm: (mm, 0, fb)),
            pl.BlockSpec((ns, _BF), lambda fb, mm: (0, fb)),
            pl.BlockSpec((ns, _BF), lambda fb, mm: (0, fb)),
            pl.BlockSpec((nb * oc, 3 * nb), lambda fb, mm: (0, 0)),
            pl.BlockSpec((oc, _CH), lambda fb, mm: (0, 0)),
        ],
        out_specs=pl.BlockSpec((1, _BF, oc), lambda fb, mm: (mm, fb, 0)),
        out_shape=jax.ShapeDtypeStruct((m, f, oc), jnp.float32),
    )(xt, btt, gtt, wt, bb)
    return out_t


# speedup vs baseline: 1.3010x; 1.3010x over previous
import functools

import jax
import jax.numpy as jnp
from jax.experimental import pallas as pl
from jax.experimental.pallas import tpu as pltpu

_BF = 6400
_CH = 128


def _cs_kernel(xt_ref, bt_ref, gt_ref, wt_ref, bb_ref, out_ref,
               *, ns: int, oc: int, nn: int):
    qt = jnp.dot(wt_ref[...], xt_ref[0],
                 preferred_element_type=jnp.float32).astype(jnp.bfloat16)
    bf = qt.shape[1]
    nk = ns // nn
    for c in range(0, bf, _CH):
        w = min(_CH, bf - c)
        acc = None
        for n in range(nn):
            q0 = qt[(3 * n + 0) * oc:(3 * n + 1) * oc, c:c + w]
            d1 = qt[(3 * n + 1) * oc:(3 * n + 2) * oc, c:c + w]
            d2 = qt[(3 * n + 2) * oc:(3 * n + 3) * oc, c:c + w]
            accn = None
            for k in range(nk):
                s = n + nn * k
                x = (bt_ref[s:s + 1, c:c + w] * d1
                     + gt_ref[s:s + 1, c:c + w] * d2)
                accn = x if accn is None else jnp.maximum(accn, x)
            an = accn + q0
            acc = an if acc is None else jnp.maximum(acc, an)
        qc = qt[3 * nn * oc:(3 * nn + 1) * oc, c:c + w]
        out_ref[0, :, c:c + w] = jnp.maximum(
            (acc - qc).astype(jnp.float32) + bb_ref[:, :w], 0.0)


def kernel(ring_n, neighbor_corners, centers, alpha, beta, gamma, W, b):
    del ring_n, alpha
    m, f, nn = neighbor_corners.shape[:3]
    ns = beta.shape[1]
    oc = W.shape[0]
    nb = 3 * nn + 1
    x30 = jnp.concatenate(
        [neighbor_corners.reshape(m, f, nn * 9), centers], axis=-1)
    xt = x30.transpose(0, 2, 1)
    btt = beta.T.astype(jnp.bfloat16)
    gtt = gamma.T.astype(jnp.bfloat16)

    tm = jnp.zeros((nb, nb), jnp.float32)
    for n in range(nn):
        tm = tm.at[3 * n + 0, 3 * n + 0].set(1.0)
        tm = tm.at[3 * n + 0, 3 * n + 1].set(-1.0)
        tm = tm.at[3 * n + 0, 3 * n + 2].set(-1.0)
        tm = tm.at[3 * n + 1, 3 * n + 1].set(1.0)
        tm = tm.at[3 * n + 2, 3 * n + 2].set(1.0)
    tm = tm.at[nb - 1, nb - 1].set(1.0)
    wt = jnp.kron(tm, W.T).T
    bb = jnp.broadcast_to(b[:, None], (oc, _CH))

    grid = ((f + _BF - 1) // _BF, m)
    out_t = pl.pallas_call(
        functools.partial(_cs_kernel, ns=ns, oc=oc, nn=nn),
        grid=grid,
        in_specs=[
            pl.BlockSpec((1, 3 * nb, _BF), lambda fb, mm: (mm, 0, fb)),
            pl.BlockSpec((ns, _BF), lambda fb, mm: (0, fb)),
            pl.BlockSpec((ns, _BF), lambda fb, mm: (0, fb)),
            pl.BlockSpec((nb * oc, 3 * nb), lambda fb, mm: (0, 0)),
            pl.BlockSpec((oc, _CH), lambda fb, mm: (0, 0)),
        ],
        out_specs=pl.BlockSpec((1, oc, _BF), lambda fb, mm: (mm, 0, fb)),
        out_shape=jax.ShapeDtypeStruct((m, oc, f), jnp.float32),
    )(xt, btt, gtt, wt, bb)
    return out_t.transpose(0, 2, 1)
